# baseline (speedup 1.0000x reference)
import functools

import jax
import jax.numpy as jnp
from jax import lax
from jax.experimental import pallas as pl
from jax.experimental.pallas import tpu as pltpu

_B, _C, _NH, _HD, _H, _W = 8, 96, 4, 24, 32, 32
_P = _H * _W
_BN_EPS = 1e-5
_LN_EPS = 1e-6
_SCALE = _HD ** (-0.5)


def _offsets(kh, kw, ph, pw, dh=1, dw=1):
    return [(i * dh - ph, j * dw - pw) for i in range(kh) for j in range(kw)]


_CONVS = [
    ("b0_0", (1, 1, 0, 0, 1, 1)),
    ("b1_0", (1, 1, 0, 0, 1, 1)),
    ("b1_1", (1, 3, 0, 1, 1, 1)),
    ("b1_2", (3, 1, 1, 0, 1, 1)),
    ("b1_3", (3, 3, 3, 3, 3, 3)),
    ("b2_0", (1, 1, 0, 0, 1, 1)),
    ("b2_1", (1, 5, 0, 2, 1, 1)),
    ("b2_2", (5, 1, 2, 0, 1, 1)),
    ("b2_3", (3, 3, 5, 5, 5, 5)),
    ("b3_0", (1, 1, 0, 0, 1, 1)),
    ("b3_1", (1, 7, 0, 3, 1, 1)),
    ("b3_2", (7, 1, 3, 0, 1, 1)),
    ("b3_3", (3, 3, 7, 7, 7, 7)),
    ("cat", (3, 3, 1, 1, 1, 1)),
    ("res", (1, 1, 0, 0, 1, 1)),
]
_CONV_OFFS = [_offsets(*spec) for _, spec in _CONVS]


def _gelu(v):
    return 0.5 * v * (1.0 + lax.erf(v * (2.0 ** -0.5)))


def _block_kernel(x_ref, cbias_ref, vecs_ref, qbh_ref, kbh_ref, vbh_ref,
                  wqh_ref, wkh_ref, wvh_ref, wph_ref,
                  mw1_ref, mb1_ref, mw2_ref, *rest):
    wrefs = rest[:-1]
    out_ref = rest[-1]
    f32 = jnp.float32
    bf16 = jnp.bfloat16
    xb = x_ref[0]

    pid = lax.broadcasted_iota(jnp.int32, (_P, 1), 0)
    hh = pid // _W
    ww = pid % _W
    mask_cache = {}

    def shifted(a, oy, ox):
        if (oy, ox) == (0, 0):
            return a
        s = oy * _W + ox
        r = jnp.concatenate([a[s:], a[:s]], axis=0)
        key = (oy, ox, a.dtype)
        if key not in mask_cache:
            m = ((hh + oy >= 0) & (hh + oy < _H)
                 & (ww + ox >= 0) & (ww + ox < _W))
            mask_cache[key] = m.astype(a.dtype)
        return r * mask_cache[key]

    def mm(a, w):
        return jnp.dot(a.astype(bf16), w, preferred_element_type=f32)

    def tree_sum(parts):
        parts = list(parts)
        while len(parts) > 1:
            nxt = [parts[i] + parts[i + 1] for i in range(0, len(parts) - 1, 2)]
            if len(parts) % 2:
                nxt.append(parts[-1])
            parts = nxt
        return parts[0]

    def conv(a, ci):
        ab = a.astype(bf16)
        parts = [jnp.dot(shifted(ab, oy, ox), wrefs[ci][t],
                         preferred_element_type=f32)
                 for t, (oy, ox) in enumerate(_CONV_OFFS[ci])]
        return tree_sum(parts) + cbias_ref[ci]

    def basic(a, ci):
        return _gelu(conv(a, ci))

    def ln(v, wi, bi):
        u = jnp.mean(v, axis=1, keepdims=True)
        d = v - u
        var = jnp.mean(d * d, axis=1, keepdims=True)
        return d * lax.rsqrt(var + _LN_EPS) * vecs_ref[wi] + vecs_ref[bi]

    x0 = basic(xb, 0)
    x1 = basic(xb, 1)
    x2 = basic(xb, 5)
    x3 = basic(xb, 9)
    res = basic(xb, 14)
    x1 = basic(x1, 2)
    x2 = basic(x2, 6)
    x3 = basic(x3, 10)
    x1 = basic(x1, 3)
    x2 = basic(x2, 7)
    x3 = basic(x3, 11)
    x1 = basic(x1, 4)
    x2 = basic(x2, 8)
    x3 = basic(x3, 12)

    branches = tuple(a.astype(bf16) for a in (x0, x1, x2, x3))
    parts = [jnp.dot(shifted(a, oy, ox), wrefs[13][t, br],
                     preferred_element_type=f32)
             for t, (oy, ox) in enumerate(_CONV_OFFS[13])
             for br, a in enumerate(branches)]
    xc = _gelu(tree_sum(parts) + cbias_ref[13])
    y = _gelu(xc + res)
    sc = y

    yl = ln(y, 0, 1)
    ylb = yl.astype(bf16)
    y2 = jnp.zeros((_P, _C), f32) + vecs_ref[4]
    qs = [jnp.dot(ylb, wqh_ref[h], preferred_element_type=f32) + qbh_ref[h]
          for h in range(_NH)]
    ks = [jnp.dot(ylb, wkh_ref[h], preferred_element_type=f32) + kbh_ref[h]
          for h in range(_NH)]
    vs = [jnp.dot(ylb, wvh_ref[h], preferred_element_type=f32) + vbh_ref[h]
          for h in range(_NH)]
    ones_col = jnp.ones((_P, 1), f32)
    vaugs = [jnp.concatenate([vs[h], ones_col], axis=1).astype(bf16)
             for h in range(_NH)]
    logits = [lax.dot_general(
        qs[h].astype(bf16), ks[h].astype(bf16), (((1,), (1,)), ((), ())),
        preferred_element_type=f32) for h in range(_NH)]
    es = [jnp.exp2(logits[h]).astype(bf16) for h in range(_NH)]
    o_augs = [jnp.dot(es[h], vaugs[h], preferred_element_type=f32)
              for h in range(_NH)]
    for h in range(_NH):
        o = o_augs[h][:, :_HD] / o_augs[h][:, _HD:]
        y2 = y2 + mm(o, wph_ref[h])
    y = y2 + sc

    m = ln(y, 2, 3)
    m = mm(m, mw1_ref[...]) + mb1_ref[...]
    m = mm(m, mw2_ref[...]) + vecs_ref[5]
    out_ref[0] = y + m


def kernel(x, params):
    p = params
    f32 = jnp.float32
    bf16 = jnp.bfloat16

    xs = x.reshape(_B, _C, _P).transpose(0, 2, 1)

    conv_ws = []
    cbias = []
    for name, (kh, kw, _, _, _, _) in _CONVS:
        cp = p[name]
        s = cp['g'] * lax.rsqrt(cp['rv'] + _BN_EPS)
        w = cp['w'].transpose(2, 3, 1, 0).reshape(kh * kw, -1, _C)
        w = w * s[None, None, :]
        if name == "cat":
            w = w.reshape(kh * kw, 4, _C, _C)
        conv_ws.append(w.astype(bf16))
        cbias.append(cp['b'] - cp['rm'] * s)
    cbias = jnp.stack(cbias).reshape(len(_CONVS), 1, _C)

    vecs = jnp.stack([p['ln1_w'], p['ln1_b'], p['ln2_w'], p['ln2_b'],
                      p['bp'], p['mlp_b2']]).reshape(6, 1, _C)

    def head_w(w):
        return w[:, :, 0, 0].T.reshape(_C, _NH, _HD).transpose(1, 0, 2)

    log2e = 1.4426950408889634
    wqh = (head_w(p['wq']) * (_SCALE * log2e)).astype(bf16)
    wvh = head_w(p['wv']).astype(bf16)
    wkh = head_w(p['wk']).astype(bf16)
    wph = p['wp'][:, :, 0, 0].T.reshape(_NH, _HD, _C).astype(bf16)
    qbh = p['bq'].reshape(_NH, 1, _HD) * (_SCALE * log2e)
    vbh = p['bv'].reshape(_NH, 1, _HD)
    kbh = p['bk'].reshape(_NH, 1, _HD)

    mw1 = p['mlp_w1'][:, :, 0, 0].T.astype(bf16)
    mb1 = p['mlp_b1'].reshape(1, 2 * _C)
    mw2 = p['mlp_w2'][:, :, 0, 0].T.astype(bf16)

    fixed = [cbias, vecs, qbh, kbh, vbh, wqh, wkh, wvh, wph, mw1, mb1, mw2]
    operands = [xs] + fixed + conv_ws

    def full_spec(a):
        nd = a.ndim
        return pl.BlockSpec(a.shape, lambda b, _n=nd: (0,) * _n)

    in_specs = [pl.BlockSpec((1, _P, _C), lambda b: (b, 0, 0))]
    in_specs += [full_spec(a) for a in fixed + conv_ws]

    out = pl.pallas_call(
        functools.partial(_block_kernel),
        grid=(_B,),
        in_specs=in_specs,
        out_specs=pl.BlockSpec((1, _P, _C), lambda b: (b, 0, 0)),
        out_shape=jax.ShapeDtypeStruct((_B, _P, _C), f32),
        compiler_params=pltpu.CompilerParams(
            dimension_semantics=("parallel",)),
    )(*operands)

    return out.transpose(0, 2, 1).reshape(_B, _C, _H, _W)


# speedup vs baseline: 1.0411x; 1.0411x over previous
import functools

import jax
import jax.numpy as jnp
from jax import lax
from jax.experimental import pallas as pl
from jax.experimental.pallas import tpu as pltpu

_B, _C, _NH, _HD, _H, _W = 8, 96, 4, 24, 32, 32
_P = _H * _W
_BN_EPS = 1e-5
_LN_EPS = 1e-6
_SCALE = _HD ** (-0.5)


def _offsets(kh, kw, ph, pw, dh=1, dw=1):
    return [(i * dh - ph, j * dw - pw) for i in range(kh) for j in range(kw)]


_CONVS = [
    ("b0_0", (1, 1, 0, 0, 1, 1)),
    ("b1_0", (1, 1, 0, 0, 1, 1)),
    ("b1_1", (1, 3, 0, 1, 1, 1)),
    ("b1_2", (3, 1, 1, 0, 1, 1)),
    ("b1_3", (3, 3, 3, 3, 3, 3)),
    ("b2_0", (1, 1, 0, 0, 1, 1)),
    ("b2_1", (1, 5, 0, 2, 1, 1)),
    ("b2_2", (5, 1, 2, 0, 1, 1)),
    ("b2_3", (3, 3, 5, 5, 5, 5)),
    ("b3_0", (1, 1, 0, 0, 1, 1)),
    ("b3_1", (1, 7, 0, 3, 1, 1)),
    ("b3_2", (7, 1, 3, 0, 1, 1)),
    ("b3_3", (3, 3, 7, 7, 7, 7)),
    ("cat", (3, 3, 1, 1, 1, 1)),
    ("res", (1, 1, 0, 0, 1, 1)),
]
_CONV_OFFS = [_offsets(*spec) for _, spec in _CONVS]


def _gelu(v):
    return 0.5 * v * (1.0 + lax.erf(v * (2.0 ** -0.5)))


def _block_kernel(x_ref, cbias_ref, vecs_ref,
                  aq_ref, ak_ref, av_ref, sq_ref, sk_ref, sv_ref,
                  c0q_ref, c0k_ref, c0v_ref, wph_ref,
                  a1_ref, s1_ref, c01_ref, mw2_ref, *rest):
    wrefs = rest[:-1]
    out_ref = rest[-1]
    f32 = jnp.float32
    bf16 = jnp.bfloat16
    xb = x_ref[0]

    pid = lax.broadcasted_iota(jnp.int32, (_P, 1), 0)
    hh = pid // _W
    ww = pid % _W
    mask_cache = {}

    def shifted(a, oy, ox):
        if (oy, ox) == (0, 0):
            return a
        s = oy * _W + ox
        r = jnp.concatenate([a[s:], a[:s]], axis=0)
        key = (oy, ox, a.dtype)
        if key not in mask_cache:
            m = ((hh + oy >= 0) & (hh + oy < _H)
                 & (ww + ox >= 0) & (ww + ox < _W))
            mask_cache[key] = m.astype(a.dtype)
        return r * mask_cache[key]

    def mm(a, w):
        return jnp.dot(a.astype(bf16), w, preferred_element_type=f32)

    def conv(a, ci):
        ab = a.astype(bf16)
        acc = jnp.zeros((_P, _C), f32) + cbias_ref[ci]
        for t, (oy, ox) in enumerate(_CONV_OFFS[ci]):
            acc = acc + jnp.dot(shifted(ab, oy, ox), wrefs[ci][t],
                                preferred_element_type=f32)
        return acc

    def basic(a, ci):
        return _gelu(conv(a, ci))

    x0 = basic(xb, 0)
    x1 = basic(xb, 1)
    x2 = basic(xb, 5)
    x3 = basic(xb, 9)
    res = basic(xb, 14)
    x1 = basic(x1, 2)
    x2 = basic(x2, 6)
    x3 = basic(x3, 10)
    x1 = basic(x1, 3)
    x2 = basic(x2, 7)
    x3 = basic(x3, 11)
    x1 = basic(x1, 4)
    x2 = basic(x2, 8)
    x3 = basic(x3, 12)

    branches = tuple(a.astype(bf16) for a in (x0, x1, x2, x3))
    acc = jnp.zeros((_P, _C), f32) + cbias_ref[13]
    for t, (oy, ox) in enumerate(_CONV_OFFS[13]):
        for br, a in enumerate(branches):
            acc = acc + jnp.dot(shifted(a, oy, ox), wrefs[13][t, br],
                                preferred_element_type=f32)
    xc = _gelu(acc)
    y = _gelu(xc + res)
    sc = y

    u = jnp.mean(y, axis=1, keepdims=True)
    ey2 = jnp.mean(y * y, axis=1, keepdims=True)
    yb2 = y.astype(bf16)
    yqA = [jnp.dot(yb2, aq_ref[h], preferred_element_type=f32)
           for h in range(_NH)]
    ykA = [jnp.dot(yb2, ak_ref[h], preferred_element_type=f32)
           for h in range(_NH)]
    yvA = [jnp.dot(yb2, av_ref[h], preferred_element_type=f32)
           for h in range(_NH)]
    rstd = lax.rsqrt(ey2 - u * u + _LN_EPS)
    ru = rstd * u
    y2 = jnp.zeros((_P, _C), f32) + vecs_ref[0]
    qs = [rstd * yqA[h] - ru * sq_ref[h] + c0q_ref[h] for h in range(_NH)]
    ks = [rstd * ykA[h] - ru * sk_ref[h] + c0k_ref[h] for h in range(_NH)]
    vs = [rstd * yvA[h] - ru * sv_ref[h] + c0v_ref[h] for h in range(_NH)]
    ones_col = jnp.ones((_P, 1), f32)
    vaugs = [jnp.concatenate([vs[h], ones_col], axis=1).astype(bf16)
             for h in range(_NH)]
    logits = [lax.dot_general(
        qs[h].astype(bf16), ks[h].astype(bf16), (((1,), (1,)), ((), ())),
        preferred_element_type=f32) for h in range(_NH)]
    es = [jnp.exp2(logits[h]).astype(bf16) for h in range(_NH)]
    o_augs = [jnp.dot(es[h], vaugs[h], preferred_element_type=f32)
              for h in range(_NH)]
    for h in range(_NH):
        o = o_augs[h][:, :_HD] / o_augs[h][:, _HD:]
        y2 = y2 + mm(o, wph_ref[h])
    y = y2 + sc

    u2 = jnp.mean(y, axis=1, keepdims=True)
    ey22 = jnp.mean(y * y, axis=1, keepdims=True)
    m1raw = jnp.dot(y.astype(bf16), a1_ref[...],
                    preferred_element_type=f32)
    rstd2 = lax.rsqrt(ey22 - u2 * u2 + _LN_EPS)
    m1 = rstd2 * m1raw - (rstd2 * u2) * s1_ref[...] + c01_ref[...]
    m2 = mm(m1, mw2_ref[...]) + vecs_ref[1]
    out_ref[0] = y + m2


def kernel(x, params):
    p = params
    f32 = jnp.float32
    bf16 = jnp.bfloat16

    xs = x.reshape(_B, _C, _P).transpose(0, 2, 1)

    conv_ws = []
    cbias = []
    for name, (kh, kw, _, _, _, _) in _CONVS:
        cp = p[name]
        s = cp['g'] * lax.rsqrt(cp['rv'] + _BN_EPS)
        w = cp['w'].transpose(2, 3, 1, 0).reshape(kh * kw, -1, _C)
        w = w * s[None, None, :]
        if name == "cat":
            w = w.reshape(kh * kw, 4, _C, _C)
        conv_ws.append(w.astype(bf16))
        cbias.append(cp['b'] - cp['rm'] * s)
    cbias = jnp.stack(cbias).reshape(len(_CONVS), 1, _C)

    vecs = jnp.stack([p['bp'], p['mlp_b2']]).reshape(2, 1, _C)

    def head_w(w):
        return w[:, :, 0, 0].T.reshape(_C, _NH, _HD).transpose(1, 0, 2)

    log2e = 1.4426950408889634
    lnw1, lnb1 = p['ln1_w'], p['ln1_b']

    def fold_ln(w_h, bias, scale):
        a = lnw1[None, :, None] * w_h * scale
        s = a.sum(axis=1, keepdims=True)
        c0 = (jnp.einsum('c,hcd->hd', lnb1, w_h) * scale
              + bias.reshape(_NH, _HD) * scale).reshape(_NH, 1, _HD)
        return a.astype(bf16), s, c0

    aq, sq, c0q = fold_ln(head_w(p['wq']), p['bq'], _SCALE * log2e)
    ak, sk, c0k = fold_ln(head_w(p['wk']), p['bk'], 1.0)
    av, sv, c0v = fold_ln(head_w(p['wv']), p['bv'], 1.0)
    wph = p['wp'][:, :, 0, 0].T.reshape(_NH, _HD, _C).astype(bf16)

    w1 = p['mlp_w1'][:, :, 0, 0].T
    a1f = p['ln2_w'][:, None] * w1
    a1 = a1f.astype(bf16)
    s1 = a1f.sum(axis=0, keepdims=True)
    c01 = (p['ln2_b'] @ w1 + p['mlp_b1']).reshape(1, 2 * _C)
    mw2 = p['mlp_w2'][:, :, 0, 0].T.astype(bf16)

    fixed = [cbias, vecs, aq, ak, av, sq, sk, sv, c0q, c0k, c0v, wph,
             a1, s1, c01, mw2]
    operands = [xs] + fixed + conv_ws

    def full_spec(a):
        nd = a.ndim
        return pl.BlockSpec(a.shape, lambda b, _n=nd: (0,) * _n)

    in_specs = [pl.BlockSpec((1, _P, _C), lambda b: (b, 0, 0))]
    in_specs += [full_spec(a) for a in fixed + conv_ws]

    out = pl.pallas_call(
        functools.partial(_block_kernel),
        grid=(_B,),
        in_specs=in_specs,
        out_specs=pl.BlockSpec((1, _P, _C), lambda b: (b, 0, 0)),
        out_shape=jax.ShapeDtypeStruct((_B, _P, _C), f32),
        compiler_params=pltpu.CompilerParams(
            dimension_semantics=("parallel",)),
    )(*operands)

    return out.transpose(0, 2, 1).reshape(_B, _C, _H, _W)


# speedup vs baseline: 1.0527x; 1.0112x over previous
import functools

import jax
import jax.numpy as jnp
from jax import lax
from jax.experimental import pallas as pl
from jax.experimental.pallas import tpu as pltpu

_B, _C, _NH, _HD, _H, _W = 8, 96, 4, 24, 32, 32
_P = _H * _W
_BN_EPS = 1e-5
_LN_EPS = 1e-6
_SCALE = _HD ** (-0.5)


def _offsets(kh, kw, ph, pw, dh=1, dw=1):
    return [(i * dh - ph, j * dw - pw) for i in range(kh) for j in range(kw)]


_CONVS = [
    ("b0_0", (1, 1, 0, 0, 1, 1)),
    ("b1_0", (1, 1, 0, 0, 1, 1)),
    ("b1_1", (1, 3, 0, 1, 1, 1)),
    ("b1_2", (3, 1, 1, 0, 1, 1)),
    ("b1_3", (3, 3, 3, 3, 3, 3)),
    ("b2_0", (1, 1, 0, 0, 1, 1)),
    ("b2_1", (1, 5, 0, 2, 1, 1)),
    ("b2_2", (5, 1, 2, 0, 1, 1)),
    ("b2_3", (3, 3, 5, 5, 5, 5)),
    ("b3_0", (1, 1, 0, 0, 1, 1)),
    ("b3_1", (1, 7, 0, 3, 1, 1)),
    ("b3_2", (7, 1, 3, 0, 1, 1)),
    ("b3_3", (3, 3, 7, 7, 7, 7)),
    ("cat", (3, 3, 1, 1, 1, 1)),
    ("res", (1, 1, 0, 0, 1, 1)),
]
_CONV_OFFS = [_offsets(*spec) for _, spec in _CONVS]


def _gelu(v):
    return 0.5 * v * (1.0 + lax.erf(v * (2.0 ** -0.5)))


def _block_kernel(x_ref, cbias_ref, vecs_ref,
                  aq_ref, ak_ref, av_ref, sq_ref, sk_ref, sv_ref,
                  c0q_ref, c0k_ref, c0v_ref, wph_ref,
                  a1_ref, s1_ref, c01_ref, mw2_ref, *rest):
    wrefs = rest[:-1]
    out_ref = rest[-1]
    f32 = jnp.float32
    bf16 = jnp.bfloat16
    xb = x_ref[0]

    pid = lax.broadcasted_iota(jnp.int32, (_P, 1), 0)
    ww = pid % _W
    mask_cache = {}

    def shifted(a, oy, ox):
        if (oy, ox) == (0, 0):
            return a
        s = oy * _W + ox
        z = jnp.zeros((abs(s), a.shape[1]), a.dtype)
        if s > 0:
            r = jnp.concatenate([a[s:], z], axis=0)
        else:
            r = jnp.concatenate([z, a[:s]], axis=0)
        if ox == 0:
            return r
        key = (ox, a.dtype)
        if key not in mask_cache:
            m = (ww + ox >= 0) & (ww + ox < _W)
            mask_cache[key] = m.astype(a.dtype)
        return r * mask_cache[key]

    def mm(a, w):
        return jnp.dot(a.astype(bf16), w, preferred_element_type=f32)

    def conv(a, ci):
        ab = a.astype(bf16)
        acc = jnp.zeros((_P, _C), f32) + cbias_ref[ci]
        for t, (oy, ox) in enumerate(_CONV_OFFS[ci]):
            acc = acc + jnp.dot(shifted(ab, oy, ox), wrefs[ci][t],
                                preferred_element_type=f32)
        return acc

    def basic(a, ci):
        return _gelu(conv(a, ci))

    x0 = basic(xb, 0)
    x1 = basic(xb, 1)
    x2 = basic(xb, 5)
    x3 = basic(xb, 9)
    res = basic(xb, 14)
    x1 = basic(x1, 2)
    x2 = basic(x2, 6)
    x3 = basic(x3, 10)
    x1 = basic(x1, 3)
    x2 = basic(x2, 7)
    x3 = basic(x3, 11)
    x1 = basic(x1, 4)
    x2 = basic(x2, 8)
    x3 = basic(x3, 12)

    branches = tuple(a.astype(bf16) for a in (x0, x1, x2, x3))
    acc = jnp.zeros((_P, _C), f32) + cbias_ref[13]
    for t, (oy, ox) in enumerate(_CONV_OFFS[13]):
        for br, a in enumerate(branches):
            acc = acc + jnp.dot(shifted(a, oy, ox), wrefs[13][t, br],
                                preferred_element_type=f32)
    xc = _gelu(acc)
    y = _gelu(xc + res)
    sc = y

    u = jnp.mean(y, axis=1, keepdims=True)
    ey2 = jnp.mean(y * y, axis=1, keepdims=True)
    yb2 = y.astype(bf16)
    yqA = [jnp.dot(yb2, aq_ref[h], preferred_element_type=f32)
           for h in range(_NH)]
    ykA = [jnp.dot(yb2, ak_ref[h], preferred_element_type=f32)
           for h in range(_NH)]
    yvA = [jnp.dot(yb2, av_ref[h], preferred_element_type=f32)
           for h in range(_NH)]
    rstd = lax.rsqrt(ey2 - u * u + _LN_EPS)
    ru = rstd * u
    y2 = jnp.zeros((_P, _C), f32) + vecs_ref[0]
    qs = [rstd * yqA[h] - ru * sq_ref[h] + c0q_ref[h] for h in range(_NH)]
    ks = [rstd * ykA[h] - ru * sk_ref[h] + c0k_ref[h] for h in range(_NH)]
    vs = [rstd * yvA[h] - ru * sv_ref[h] + c0v_ref[h] for h in range(_NH)]
    ones_col = jnp.ones((_P, 1), f32)
    vaugs = [jnp.concatenate([vs[h], ones_col], axis=1).astype(bf16)
             for h in range(_NH)]
    logits = [lax.dot_general(
        qs[h].astype(bf16), ks[h].astype(bf16), (((1,), (1,)), ((), ())),
        preferred_element_type=f32) for h in range(_NH)]
    es = [jnp.exp2(logits[h]).astype(bf16) for h in range(_NH)]
    o_augs = [jnp.dot(es[h], vaugs[h], preferred_element_type=f32)
              for h in range(_NH)]
    for h in range(_NH):
        o = o_augs[h] / o_augs[h][:, _HD:]
        y2 = y2 + mm(o, wph_ref[h])
    y = y2 + sc

    u2 = jnp.mean(y, axis=1, keepdims=True)
    ey22 = jnp.mean(y * y, axis=1, keepdims=True)
    m1raw = jnp.dot(y.astype(bf16), a1_ref[...],
                    preferred_element_type=f32)
    rstd2 = lax.rsqrt(ey22 - u2 * u2 + _LN_EPS)
    m1 = rstd2 * m1raw - (rstd2 * u2) * s1_ref[...] + c01_ref[...]
    m2 = mm(m1, mw2_ref[...]) + vecs_ref[1]
    out_ref[0] = y + m2


def kernel(x, params):
    p = params
    f32 = jnp.float32
    bf16 = jnp.bfloat16

    xs = x.reshape(_B, _C, _P).transpose(0, 2, 1)

    conv_ws = []
    cbias = []
    for name, (kh, kw, _, _, _, _) in _CONVS:
        cp = p[name]
        s = cp['g'] * lax.rsqrt(cp['rv'] + _BN_EPS)
        w = cp['w'].transpose(2, 3, 1, 0).reshape(kh * kw, -1, _C)
        w = w * s[None, None, :]
        if name == "cat":
            w = w.reshape(kh * kw, 4, _C, _C)
        conv_ws.append(w.astype(bf16))
        cbias.append(cp['b'] - cp['rm'] * s)
    cbias = jnp.stack(cbias).reshape(len(_CONVS), 1, _C)

    vecs = jnp.stack([p['bp'], p['mlp_b2']]).reshape(2, 1, _C)

    def head_w(w):
        return w[:, :, 0, 0].T.reshape(_C, _NH, _HD).transpose(1, 0, 2)

    log2e = 1.4426950408889634
    lnw1, lnb1 = p['ln1_w'], p['ln1_b']

    def fold_ln(w_h, bias, scale):
        a = lnw1[None, :, None] * w_h * scale
        s = a.sum(axis=1, keepdims=True)
        c0 = (jnp.einsum('c,hcd->hd', lnb1, w_h) * scale
              + bias.reshape(_NH, _HD) * scale).reshape(_NH, 1, _HD)
        return a.astype(bf16), s, c0

    aq, sq, c0q = fold_ln(head_w(p['wq']), p['bq'], _SCALE * log2e)
    ak, sk, c0k = fold_ln(head_w(p['wk']), p['bk'], 1.0)
    av, sv, c0v = fold_ln(head_w(p['wv']), p['bv'], 1.0)
    wph = p['wp'][:, :, 0, 0].T.reshape(_NH, _HD, _C)
    wph = jnp.concatenate(
        [wph, jnp.zeros((_NH, 1, _C), f32)], axis=1).astype(bf16)

    w1 = p['mlp_w1'][:, :, 0, 0].T
    a1f = p['ln2_w'][:, None] * w1
    a1 = a1f.astype(bf16)
    s1 = a1f.sum(axis=0, keepdims=True)
    c01 = (p['ln2_b'] @ w1 + p['mlp_b1']).reshape(1, 2 * _C)
    mw2 = p['mlp_w2'][:, :, 0, 0].T.astype(bf16)

    fixed = [cbias, vecs, aq, ak, av, sq, sk, sv, c0q, c0k, c0v, wph,
             a1, s1, c01, mw2]
    operands = [xs] + fixed + conv_ws

    def full_spec(a):
        nd = a.ndim
        return pl.BlockSpec(a.shape, lambda b, _n=nd: (0,) * _n)

    in_specs = [pl.BlockSpec((1, _P, _C), lambda b: (b, 0, 0))]
    in_specs += [full_spec(a) for a in fixed + conv_ws]

    out = pl.pallas_call(
        functools.partial(_block_kernel),
        grid=(_B,),
        in_specs=in_specs,
        out_specs=pl.BlockSpec((1, _P, _C), lambda b: (b, 0, 0)),
        out_shape=jax.ShapeDtypeStruct((_B, _P, _C), f32),
        compiler_params=pltpu.CompilerParams(
            dimension_semantics=("parallel",)),
    )(*operands)

    return out.transpose(0, 2, 1).reshape(_B, _C, _H, _W)
